# baseline (speedup 1.0000x reference)
import jax
import jax.numpy as jnp
from jax import lax
from jax.experimental import pallas as pl
from jax.experimental.pallas import tpu as pltpu
from jax.experimental.pallas import tpu_sc as plsc

N = 50000
D = 512
G = 128
LANES = 16
NUM_CORES = 2
NUM_SUBCORES = 16
NW = NUM_CORES * NUM_SUBCORES
SEGS_PER_W = G // NW
R = 64
U = 8
COL_GROUPS = D // LANES

_NEG_INF = float("-inf")


_NB = N // LANES


def _searchsorted(batch_v, t):
  def step(_, carry):
    lo, hi = carry
    active = lo < hi
    mid = (lo + hi) // 2
    vec = batch_v[pl.ds(jnp.minimum(mid, _NB - 1) * LANES, LANES)]
    found = jnp.max(vec) >= t
    lo_new = jnp.where(active & ~found, mid + 1, lo)
    hi_new = jnp.where(active & found, mid, hi)
    return lo_new, hi_new

  lo, _ = lax.fori_loop(0, 12, step, (jnp.int32(0), jnp.int32(_NB)))
  m = jnp.minimum(lo, _NB - 1)
  vec = batch_v[pl.ds(m * LANES, LANES)]
  cnt = jnp.sum(jnp.where(vec < t, 1, 0))
  return m * LANES + cnt


def _worker_id():
  return lax.axis_index("s") * NUM_CORES + lax.axis_index("c")


def _sc_pool_body(x_hbm, batch_hbm, mean_hbm, max_hbm,
                  batch_v, chunk_v, acc_sum, acc_max, bounds_s,
                  sem_a, sem_b):
  wid = _worker_id()
  g_base = wid * SEGS_PER_W

  pltpu.sync_copy(batch_hbm, batch_v)
  for j in range(SEGS_PER_W + 1):
    bounds_s[j] = _searchsorted(batch_v, g_base + j)

  def seg_body(j, _):
    g = g_base + j
    start = bounds_s[j]
    end = bounds_s[j + 1]
    cnt = end - start

    zeros = jnp.zeros((LANES,), jnp.float32)
    ninf = jnp.full((LANES,), _NEG_INF, jnp.float32)
    for k in range(COL_GROUPS):
      cs = pl.ds(k * LANES, LANES)
      acc_sum[cs] = zeros
      acc_max[cs] = ninf

    a_start = (start // 8) * 8
    nc = (end - a_start + R - 1) // R

    def copy(i, half, sem):
      s_i = a_start + i * R
      o_i = jnp.minimum(s_i, N - R)
      dst = chunk_v.at[pl.ds(half * R, R)]
      return pltpu.make_async_copy(x_hbm.at[pl.ds(o_i, R)], dst, sem)

    @pl.when(nc > 0)
    def _():
      copy(0, 0, sem_a).start()

    def chunk_body(i, _):
      even = (i % 2) == 0

      @pl.when(even)
      def _():
        copy(i, 0, sem_a).wait()

      @pl.when(~even)
      def _():
        copy(i, 1, sem_b).wait()

      nxt = i + 1 < nc

      @pl.when(nxt & even)
      def _():
        copy(i + 1, 1, sem_b).start()

      @pl.when(nxt & ~even)
      def _():
        copy(i + 1, 0, sem_a).start()

      base = (i % 2) * R
      s_i = a_start + i * R
      o_i = jnp.minimum(s_i, N - R)
      j_lo = jnp.maximum(s_i, start) - o_i + base
      j_hi = jnp.minimum(s_i + R, end) - o_i + base
      nfull = (j_hi - j_lo) // U

      for k in range(COL_GROUPS):
        cs = pl.ds(k * LANES, LANES)

        def blk(ib, c, cs=cs):
          s, m = c[0], c[1]
          vals = list(c[2:])
          rb = j_lo + (ib + 1) * U
          nxt = [chunk_v[rb + u, cs] for u in range(U)]
          v, w = vals, vals
          while len(v) > 1:
            v = [v[2 * t] + v[2 * t + 1] for t in range(len(v) // 2)]
            w = [jnp.maximum(w[2 * t], w[2 * t + 1]) for t in range(len(w) // 2)]
          return (s + v[0], jnp.maximum(m, w[0])) + tuple(nxt)

        def tail(jr, c, cs=cs):
          s, m = c
          v = chunk_v[jr, cs]
          return s + v, jnp.maximum(m, v)

        first = tuple(chunk_v[j_lo + u, cs] for u in range(U))
        res = lax.fori_loop(0, nfull, blk,
                            (acc_sum[cs], acc_max[cs]) + first)
        s2, m2 = lax.fori_loop(j_lo + nfull * U, j_hi, tail,
                               (res[0], res[1]))
        acc_sum[cs] = s2
        acc_max[cs] = m2

      return 0

    lax.fori_loop(0, nc, chunk_body, 0)

    denom = jnp.maximum(cnt.astype(jnp.float32), 1.0)
    inv = jnp.ones((LANES,), jnp.float32) / lax.broadcast(denom, (LANES,))
    for k in range(COL_GROUPS):
      cs = pl.ds(k * LANES, LANES)
      acc_sum[cs] = acc_sum[cs] * inv
    pltpu.sync_copy(acc_sum, mean_hbm.at[pl.ds(g * D, D)])
    pltpu.sync_copy(acc_max, max_hbm.at[pl.ds(g * D, D)])
    return 0

  lax.fori_loop(0, SEGS_PER_W, seg_body, 0)


_sc_pool = pl.kernel(
    _sc_pool_body,
    out_type=(
        jax.ShapeDtypeStruct((G * D,), jnp.float32),
        jax.ShapeDtypeStruct((G * D,), jnp.float32),
    ),
    mesh=plsc.VectorSubcoreMesh(
        core_axis_name="c", subcore_axis_name="s",
        num_cores=NUM_CORES, num_subcores=NUM_SUBCORES,
    ),
    scratch_types=[
        pltpu.VMEM((N,), jnp.int32),
        pltpu.VMEM((2 * R + U, D), jnp.float32),
        pltpu.VMEM((D,), jnp.float32),
        pltpu.VMEM((D,), jnp.float32),
        pltpu.SMEM((SEGS_PER_W + 1,), jnp.int32),
        pltpu.SemaphoreType.DMA,
        pltpu.SemaphoreType.DMA,
    ],
    compiler_params=pltpu.CompilerParams(needs_layout_passes=False),
)


def _proj_body(mean_ref, max_ref, w_ref, b_ref, o_ref):
  w = w_ref[...]
  wm = w[:, 0:D] + w[:, 2 * D:3 * D] + w[:, 4 * D:5 * D]
  wx = w[:, D:2 * D] + w[:, 3 * D:4 * D] + w[:, 5 * D:6 * D]
  acc = lax.dot_general(mean_ref[...], wm, (((1,), (1,)), ((), ())),
                        preferred_element_type=jnp.float32)
  acc = acc + lax.dot_general(max_ref[...], wx, (((1,), (1,)), ((), ())),
                              preferred_element_type=jnp.float32)
  o_ref[...] = acc + b_ref[...]


_proj = pl.pallas_call(
    _proj_body,
    out_shape=jax.ShapeDtypeStruct((G, D), jnp.float32),
)


@jax.jit
def kernel(x, batch, W, b):
  mean, mx = _sc_pool(x, batch)
  return _proj(mean.reshape(G, D), mx.reshape(G, D), W, b.reshape(1, D))


# speedup vs baseline: 2.0069x; 2.0069x over previous
import jax
import jax.numpy as jnp
from jax import lax
from jax.experimental import pallas as pl
from jax.experimental.pallas import tpu as pltpu
from jax.experimental.pallas import tpu_sc as plsc

N = 50000
D = 512
G = 128
LANES = 16
NUM_CORES = 2
NUM_SUBCORES = 16
NW = NUM_CORES * NUM_SUBCORES
SEGS_PER_W = G // NW
R = 64
U = 8
COL_GROUPS = D // LANES

_NEG_INF = float("-inf")


_NB = N // LANES


def _searchsorted(batch_v, t):
  def step(_, carry):
    lo, hi = carry
    active = lo < hi
    mid = (lo + hi) // 2
    vec = batch_v[pl.ds(jnp.minimum(mid, _NB - 1) * LANES, LANES)]
    found = jnp.max(vec) >= t
    lo_new = jnp.where(active & ~found, mid + 1, lo)
    hi_new = jnp.where(active & found, mid, hi)
    return lo_new, hi_new

  lo, _ = lax.fori_loop(0, 12, step, (jnp.int32(0), jnp.int32(_NB)))
  m = jnp.minimum(lo, _NB - 1)
  vec = batch_v[pl.ds(m * LANES, LANES)]
  cnt = jnp.sum(jnp.where(vec < t, 1, 0))
  return m * LANES + cnt


def _worker_id():
  return lax.axis_index("s") * NUM_CORES + lax.axis_index("c")


def _sc_pool_body(x_hbm, batch_hbm, mean_hbm, max_hbm,
                  batch_v, chunk_v, acc_sum, acc_max, bounds_s,
                  sem_a, sem_b):
  wid = _worker_id()
  g_base = wid * SEGS_PER_W

  pltpu.sync_copy(batch_hbm, batch_v)
  for j in range(SEGS_PER_W + 1):
    bounds_s[j] = _searchsorted(batch_v, g_base + j)

  def seg_body(j, _):
    g = g_base + j
    start = bounds_s[j]
    end = bounds_s[j + 1]
    cnt = end - start

    zeros = jnp.zeros((LANES,), jnp.float32)
    ninf = jnp.full((LANES,), _NEG_INF, jnp.float32)
    for k in range(COL_GROUPS):
      cs = pl.ds(k * LANES, LANES)
      acc_sum[cs] = zeros
      acc_max[cs] = ninf

    a_start = (start // 8) * 8
    nc = (end - a_start + R - 1) // R

    def copy(i, half, sem):
      s_i = a_start + i * R
      o_i = jnp.minimum(s_i, N - R)
      dst = chunk_v.at[pl.ds(half * R, R)]
      return pltpu.make_async_copy(x_hbm.at[pl.ds(o_i, R)], dst, sem)

    @pl.when(nc > 0)
    def _():
      copy(0, 0, sem_a).start()

    def chunk_body(i, _):
      even = (i % 2) == 0

      @pl.when(even)
      def _():
        copy(i, 0, sem_a).wait()

      @pl.when(~even)
      def _():
        copy(i, 1, sem_b).wait()

      nxt = i + 1 < nc

      @pl.when(nxt & even)
      def _():
        copy(i + 1, 1, sem_b).start()

      @pl.when(nxt & ~even)
      def _():
        copy(i + 1, 0, sem_a).start()

      base = (i % 2) * R
      s_i = a_start + i * R
      o_i = jnp.minimum(s_i, N - R)
      j_lo = jnp.maximum(s_i, start) - o_i + base
      j_hi = jnp.minimum(s_i + R, end) - o_i + base
      nfull = (j_hi - j_lo) // U

      HALF = COL_GROUPS // 2
      n2 = (j_hi - j_lo) // 2

      for h in range(2):
        ks = range(h * HALF, (h + 1) * HALF)
        init = tuple(acc_sum[pl.ds(k * LANES, LANES)] for k in ks) + \
               tuple(acc_max[pl.ds(k * LANES, LANES)] for k in ks)

        def row_step(c, r, ks=ks):
          ss, mm = c[:HALF], c[HALF:]
          vs = [chunk_v[r, pl.ds(k * LANES, LANES)] for k in ks]
          return (tuple(s + v for s, v in zip(ss, vs)) +
                  tuple(jnp.maximum(m, v) for m, v in zip(mm, vs)))

        def two_rows(i2, c, ks=ks):
          r = j_lo + i2 * 2
          return row_step(row_step(c, r, ks), r + 1, ks)

        def one_row(r, c, ks=ks):
          return row_step(c, r, ks)

        res = lax.fori_loop(0, n2, two_rows, init)
        res = lax.fori_loop(j_lo + n2 * 2, j_hi, one_row, res)
        for idx, k in enumerate(ks):
          acc_sum[pl.ds(k * LANES, LANES)] = res[idx]
          acc_max[pl.ds(k * LANES, LANES)] = res[HALF + idx]

      return 0

    lax.fori_loop(0, nc, chunk_body, 0)

    denom = jnp.maximum(cnt.astype(jnp.float32), 1.0)
    inv = jnp.ones((LANES,), jnp.float32) / lax.broadcast(denom, (LANES,))
    for k in range(COL_GROUPS):
      cs = pl.ds(k * LANES, LANES)
      acc_sum[cs] = acc_sum[cs] * inv
    pltpu.sync_copy(acc_sum, mean_hbm.at[pl.ds(g * D, D)])
    pltpu.sync_copy(acc_max, max_hbm.at[pl.ds(g * D, D)])
    return 0

  lax.fori_loop(0, SEGS_PER_W, seg_body, 0)


_sc_pool = pl.kernel(
    _sc_pool_body,
    out_type=(
        jax.ShapeDtypeStruct((G * D,), jnp.float32),
        jax.ShapeDtypeStruct((G * D,), jnp.float32),
    ),
    mesh=plsc.VectorSubcoreMesh(
        core_axis_name="c", subcore_axis_name="s",
        num_cores=NUM_CORES, num_subcores=NUM_SUBCORES,
    ),
    scratch_types=[
        pltpu.VMEM((N,), jnp.int32),
        pltpu.VMEM((2 * R + U, D), jnp.float32),
        pltpu.VMEM((D,), jnp.float32),
        pltpu.VMEM((D,), jnp.float32),
        pltpu.SMEM((SEGS_PER_W + 1,), jnp.int32),
        pltpu.SemaphoreType.DMA,
        pltpu.SemaphoreType.DMA,
    ],
    compiler_params=pltpu.CompilerParams(needs_layout_passes=False),
)


def _proj_body(mean_ref, max_ref, w_ref, b_ref, o_ref):
  w = w_ref[...]
  wm = w[:, 0:D] + w[:, 2 * D:3 * D] + w[:, 4 * D:5 * D]
  wx = w[:, D:2 * D] + w[:, 3 * D:4 * D] + w[:, 5 * D:6 * D]
  acc = lax.dot_general(mean_ref[...], wm, (((1,), (1,)), ((), ())),
                        preferred_element_type=jnp.float32)
  acc = acc + lax.dot_general(max_ref[...], wx, (((1,), (1,)), ((), ())),
                              preferred_element_type=jnp.float32)
  o_ref[...] = acc + b_ref[...]


_proj = pl.pallas_call(
    _proj_body,
    out_shape=jax.ShapeDtypeStruct((G, D), jnp.float32),
)


@jax.jit
def kernel(x, batch, W, b):
  mean, mx = _sc_pool(x, batch)
  return _proj(mean.reshape(G, D), mx.reshape(G, D), W, b.reshape(1, D))


# speedup vs baseline: 2.0801x; 1.0365x over previous
import jax
import jax.numpy as jnp
from jax import lax
from jax.experimental import pallas as pl
from jax.experimental.pallas import tpu as pltpu
from jax.experimental.pallas import tpu_sc as plsc

N = 50000
D = 512
G = 128
LANES = 16
NUM_CORES = 2
NUM_SUBCORES = 16
NW = NUM_CORES * NUM_SUBCORES
SEGS_PER_W = G // NW
R = 96
U = 8
COL_GROUPS = D // LANES

_NEG_INF = float("-inf")


_NB = N // LANES


def _searchsorted(batch_v, t):
  def step(_, carry):
    lo, hi = carry
    active = lo < hi
    mid = (lo + hi) // 2
    vec = batch_v[pl.ds(jnp.minimum(mid, _NB - 1) * LANES, LANES)]
    found = jnp.max(vec) >= t
    lo_new = jnp.where(active & ~found, mid + 1, lo)
    hi_new = jnp.where(active & found, mid, hi)
    return lo_new, hi_new

  lo, _ = lax.fori_loop(0, 12, step, (jnp.int32(0), jnp.int32(_NB)))
  m = jnp.minimum(lo, _NB - 1)
  vec = batch_v[pl.ds(m * LANES, LANES)]
  cnt = jnp.sum(jnp.where(vec < t, 1, 0))
  return m * LANES + cnt


def _worker_id():
  return lax.axis_index("s") * NUM_CORES + lax.axis_index("c")


def _sc_pool_body(x_hbm, batch_hbm, mean_hbm, max_hbm,
                  acc_sum, acc_max, bounds_s, sem_a, sem_b):
  wid = _worker_id()
  g_base = wid * SEGS_PER_W

  def search_phase(batch_v):
    pltpu.sync_copy(batch_hbm, batch_v)
    for j in range(SEGS_PER_W + 1):
      bounds_s[j] = _searchsorted(batch_v, g_base + j)

  pl.run_scoped(search_phase, pltpu.VMEM((N,), jnp.int32))

  def main_phase(chunk_v):
   def seg_body(j, _):
    g = g_base + j
    start = bounds_s[j]
    end = bounds_s[j + 1]
    cnt = end - start

    zeros = jnp.zeros((LANES,), jnp.float32)
    ninf = jnp.full((LANES,), _NEG_INF, jnp.float32)
    for k in range(COL_GROUPS):
      cs = pl.ds(k * LANES, LANES)
      acc_sum[cs] = zeros
      acc_max[cs] = ninf

    a_start = (start // 8) * 8
    nc = (end - a_start + R - 1) // R

    def copy(i, half, sem):
      s_i = a_start + i * R
      o_i = jnp.minimum(s_i, N - R)
      dst = chunk_v.at[pl.ds(half * R, R)]
      return pltpu.make_async_copy(x_hbm.at[pl.ds(o_i, R)], dst, sem)

    @pl.when(nc > 0)
    def _():
      copy(0, 0, sem_a).start()

    def chunk_body(i, _):
      even = (i % 2) == 0

      @pl.when(even)
      def _():
        copy(i, 0, sem_a).wait()

      @pl.when(~even)
      def _():
        copy(i, 1, sem_b).wait()

      nxt = i + 1 < nc

      @pl.when(nxt & even)
      def _():
        copy(i + 1, 1, sem_b).start()

      @pl.when(nxt & ~even)
      def _():
        copy(i + 1, 0, sem_a).start()

      base = (i % 2) * R
      s_i = a_start + i * R
      o_i = jnp.minimum(s_i, N - R)
      j_lo = jnp.maximum(s_i, start) - o_i + base
      j_hi = jnp.minimum(s_i + R, end) - o_i + base
      nfull = (j_hi - j_lo) // U

      HALF = COL_GROUPS // 2
      n2 = (j_hi - j_lo) // 2

      for h in range(2):
        ks = range(h * HALF, (h + 1) * HALF)
        init = tuple(acc_sum[pl.ds(k * LANES, LANES)] for k in ks) + \
               tuple(acc_max[pl.ds(k * LANES, LANES)] for k in ks)

        def row_step(c, r, ks=ks):
          ss, mm = c[:HALF], c[HALF:]
          vs = [chunk_v[r, pl.ds(k * LANES, LANES)] for k in ks]
          return (tuple(s + v for s, v in zip(ss, vs)) +
                  tuple(jnp.maximum(m, v) for m, v in zip(mm, vs)))

        def two_rows(i2, c, ks=ks):
          r = j_lo + i2 * 2
          return row_step(row_step(c, r, ks), r + 1, ks)

        def one_row(r, c, ks=ks):
          return row_step(c, r, ks)

        res = lax.fori_loop(0, n2, two_rows, init)
        res = lax.fori_loop(j_lo + n2 * 2, j_hi, one_row, res)
        for idx, k in enumerate(ks):
          acc_sum[pl.ds(k * LANES, LANES)] = res[idx]
          acc_max[pl.ds(k * LANES, LANES)] = res[HALF + idx]

      return 0

    lax.fori_loop(0, nc, chunk_body, 0)

    denom = jnp.maximum(cnt.astype(jnp.float32), 1.0)
    inv = jnp.ones((LANES,), jnp.float32) / lax.broadcast(denom, (LANES,))
    for k in range(COL_GROUPS):
      cs = pl.ds(k * LANES, LANES)
      acc_sum[cs] = acc_sum[cs] * inv
    pltpu.sync_copy(acc_sum, mean_hbm.at[pl.ds(g * D, D)])
    pltpu.sync_copy(acc_max, max_hbm.at[pl.ds(g * D, D)])
    return 0

   lax.fori_loop(0, SEGS_PER_W, seg_body, 0)

  pl.run_scoped(main_phase, pltpu.VMEM((2 * R, D), jnp.float32))


_sc_pool = pl.kernel(
    _sc_pool_body,
    out_type=(
        jax.ShapeDtypeStruct((G * D,), jnp.float32),
        jax.ShapeDtypeStruct((G * D,), jnp.float32),
    ),
    mesh=plsc.VectorSubcoreMesh(
        core_axis_name="c", subcore_axis_name="s",
        num_cores=NUM_CORES, num_subcores=NUM_SUBCORES,
    ),
    scratch_types=[
        pltpu.VMEM((D,), jnp.float32),
        pltpu.VMEM((D,), jnp.float32),
        pltpu.SMEM((SEGS_PER_W + 1,), jnp.int32),
        pltpu.SemaphoreType.DMA,
        pltpu.SemaphoreType.DMA,
    ],
    compiler_params=pltpu.CompilerParams(needs_layout_passes=False),
)


def _proj_body(mean_ref, max_ref, w_ref, b_ref, o_ref):
  w = w_ref[...]
  wm = w[:, 0:D] + w[:, 2 * D:3 * D] + w[:, 4 * D:5 * D]
  wx = w[:, D:2 * D] + w[:, 3 * D:4 * D] + w[:, 5 * D:6 * D]
  acc = lax.dot_general(mean_ref[...], wm, (((1,), (1,)), ((), ())),
                        preferred_element_type=jnp.float32)
  acc = acc + lax.dot_general(max_ref[...], wx, (((1,), (1,)), ((), ())),
                              preferred_element_type=jnp.float32)
  o_ref[...] = acc + b_ref[...]


_proj = pl.pallas_call(
    _proj_body,
    out_shape=jax.ShapeDtypeStruct((G, D), jnp.float32),
)


@jax.jit
def kernel(x, batch, W, b):
  mean, mx = _sc_pool(x, batch)
  return _proj(mean.reshape(G, D), mx.reshape(G, D), W, b.reshape(1, D))


# speedup vs baseline: 2.1262x; 1.0221x over previous
import jax
import jax.numpy as jnp
from jax import lax
from jax.experimental import pallas as pl
from jax.experimental.pallas import tpu as pltpu
from jax.experimental.pallas import tpu_sc as plsc

N = 50000
D = 512
G = 128
LANES = 16
NUM_CORES = 2
NUM_SUBCORES = 16
NW = NUM_CORES * NUM_SUBCORES
SEGS_PER_W = G // NW
R = 80
NBUF = 3
U = 8
COL_GROUPS = D // LANES

_NEG_INF = float("-inf")


_NB = N // LANES


def _searchsorted(batch_v, t):
  def step(_, carry):
    lo, hi = carry
    active = lo < hi
    mid = (lo + hi) // 2
    vec = batch_v[pl.ds(jnp.minimum(mid, _NB - 1) * LANES, LANES)]
    found = jnp.max(vec) >= t
    lo_new = jnp.where(active & ~found, mid + 1, lo)
    hi_new = jnp.where(active & found, mid, hi)
    return lo_new, hi_new

  lo, _ = lax.fori_loop(0, 12, step, (jnp.int32(0), jnp.int32(_NB)))
  m = jnp.minimum(lo, _NB - 1)
  vec = batch_v[pl.ds(m * LANES, LANES)]
  cnt = jnp.sum(jnp.where(vec < t, 1, 0))
  return m * LANES + cnt


def _worker_id():
  return lax.axis_index("s") * NUM_CORES + lax.axis_index("c")


def _sc_pool_body(x_hbm, batch_hbm, mean_hbm, max_hbm,
                  acc_sum, acc_max, bounds_s, sem_a, sem_b, sem_c):
  wid = _worker_id()
  g_base = wid * SEGS_PER_W

  def search_phase(batch_v):
    pltpu.sync_copy(batch_hbm, batch_v)
    for j in range(SEGS_PER_W + 1):
      bounds_s[j] = _searchsorted(batch_v, g_base + j)

  pl.run_scoped(search_phase, pltpu.VMEM((N,), jnp.int32))

  def main_phase(chunk_v):
   def seg_body(j, _):
    g = g_base + j
    start = bounds_s[j]
    end = bounds_s[j + 1]
    cnt = end - start

    zeros = jnp.zeros((LANES,), jnp.float32)
    ninf = jnp.full((LANES,), _NEG_INF, jnp.float32)
    for k in range(COL_GROUPS):
      cs = pl.ds(k * LANES, LANES)
      acc_sum[cs] = zeros
      acc_max[cs] = ninf

    a_start = (start // 8) * 8
    nc = (end - a_start + R - 1) // R
    sems = (sem_a, sem_b, sem_c)

    def copy(i, slot, sem):
      s_i = a_start + i * R
      o_i = jnp.minimum(s_i, N - R)
      dst = chunk_v.at[pl.ds(slot * R, R)]
      return pltpu.make_async_copy(x_hbm.at[pl.ds(o_i, R)], dst, sem)

    for p in range(NBUF - 1):
      @pl.when(nc > p)
      def _(p=p):
        copy(p, p, sems[p]).start()

    def chunk_body(i, _):
      slot = i % NBUF
      for p in range(NBUF):
        @pl.when(slot == p)
        def _(p=p):
          copy(i, p, sems[p]).wait()

      nxt = i + NBUF - 1
      for p in range(NBUF):
        @pl.when((nxt < nc) & (nxt % NBUF == p))
        def _(p=p):
          copy(nxt, p, sems[p]).start()

      base = slot * R
      s_i = a_start + i * R
      o_i = jnp.minimum(s_i, N - R)
      j_lo = jnp.maximum(s_i, start) - o_i + base
      j_hi = jnp.minimum(s_i + R, end) - o_i + base
      nfull = (j_hi - j_lo) // U

      HALF = COL_GROUPS // 2
      n2 = (j_hi - j_lo) // 2

      for h in range(2):
        ks = range(h * HALF, (h + 1) * HALF)
        init = tuple(acc_sum[pl.ds(k * LANES, LANES)] for k in ks) + \
               tuple(acc_max[pl.ds(k * LANES, LANES)] for k in ks)

        def row_step(c, r, ks=ks):
          ss, mm = c[:HALF], c[HALF:]
          vs = [chunk_v[r, pl.ds(k * LANES, LANES)] for k in ks]
          return (tuple(s + v for s, v in zip(ss, vs)) +
                  tuple(jnp.maximum(m, v) for m, v in zip(mm, vs)))

        def two_rows(i2, c, ks=ks):
          r = j_lo + i2 * 2
          return row_step(row_step(c, r, ks), r + 1, ks)

        def one_row(r, c, ks=ks):
          return row_step(c, r, ks)

        res = lax.fori_loop(0, n2, two_rows, init)
        res = lax.fori_loop(j_lo + n2 * 2, j_hi, one_row, res)
        for idx, k in enumerate(ks):
          acc_sum[pl.ds(k * LANES, LANES)] = res[idx]
          acc_max[pl.ds(k * LANES, LANES)] = res[HALF + idx]

      return 0

    lax.fori_loop(0, nc, chunk_body, 0)

    denom = jnp.maximum(cnt.astype(jnp.float32), 1.0)
    inv = jnp.ones((LANES,), jnp.float32) / lax.broadcast(denom, (LANES,))
    for k in range(COL_GROUPS):
      cs = pl.ds(k * LANES, LANES)
      acc_sum[cs] = acc_sum[cs] * inv
    pltpu.sync_copy(acc_sum, mean_hbm.at[pl.ds(g * D, D)])
    pltpu.sync_copy(acc_max, max_hbm.at[pl.ds(g * D, D)])
    return 0

   lax.fori_loop(0, SEGS_PER_W, seg_body, 0)

  pl.run_scoped(main_phase, pltpu.VMEM((NBUF * R, D), jnp.float32))


_sc_pool = pl.kernel(
    _sc_pool_body,
    out_type=(
        jax.ShapeDtypeStruct((G * D,), jnp.float32),
        jax.ShapeDtypeStruct((G * D,), jnp.float32),
    ),
    mesh=plsc.VectorSubcoreMesh(
        core_axis_name="c", subcore_axis_name="s",
        num_cores=NUM_CORES, num_subcores=NUM_SUBCORES,
    ),
    scratch_types=[
        pltpu.VMEM((D,), jnp.float32),
        pltpu.VMEM((D,), jnp.float32),
        pltpu.SMEM((SEGS_PER_W + 1,), jnp.int32),
        pltpu.SemaphoreType.DMA,
        pltpu.SemaphoreType.DMA,
        pltpu.SemaphoreType.DMA,
    ],
    compiler_params=pltpu.CompilerParams(needs_layout_passes=False),
)


def _proj_body(mean_ref, max_ref, w_ref, b_ref, o_ref):
  w = w_ref[...]
  wm = w[:, 0:D] + w[:, 2 * D:3 * D] + w[:, 4 * D:5 * D]
  wx = w[:, D:2 * D] + w[:, 3 * D:4 * D] + w[:, 5 * D:6 * D]
  acc = lax.dot_general(mean_ref[...], wm, (((1,), (1,)), ((), ())),
                        preferred_element_type=jnp.float32)
  acc = acc + lax.dot_general(max_ref[...], wx, (((1,), (1,)), ((), ())),
                              preferred_element_type=jnp.float32)
  o_ref[...] = acc + b_ref[...]


_proj = pl.pallas_call(
    _proj_body,
    out_shape=jax.ShapeDtypeStruct((G, D), jnp.float32),
)


@jax.jit
def kernel(x, batch, W, b):
  mean, mx = _sc_pool(x, batch)
  return _proj(mean.reshape(G, D), mx.reshape(G, D), W, b.reshape(1, D))
